# baseline (speedup 1.0000x reference)
import functools

import jax
import jax.numpy as jnp
from jax import lax
from jax.experimental import pallas as pl
from jax.experimental.pallas import tpu as pltpu
from jax.experimental.pallas import tpu_sc as plsc

_NCLS = 1000
_B = 16384
_HALF = _B // 2
_BLK = 1024

_NPAD = 1024


def _tc_body(yh_ref, y_ref, loss_ref, cls_ref):
    yh = yh_ref[...]
    yv = y_ref[...]
    m = jnp.max(yh, axis=1, keepdims=True)
    lse = jnp.log(jnp.sum(jnp.exp(yh - m), axis=1, keepdims=True))
    sy = jnp.sum(yv, axis=1)
    syh = jnp.sum(yv * yh, axis=1)
    loss_ref[...] = sy * (m[:, 0] + lse[:, 0]) - syh
    ym = jnp.max(yv, axis=1, keepdims=True)
    colid = lax.broadcasted_iota(jnp.int32, yv.shape, 1)
    cls_ref[...] = jnp.min(jnp.where(yv == ym, colid, _NCLS), axis=1)


def _tc_loss(y_hat, y, row0):
    off = row0 // _BLK
    return pl.pallas_call(
        _tc_body,
        grid=(_HALF // _BLK,),
        in_specs=[
            pl.BlockSpec((_BLK, _NCLS), lambda i: (i + off, 0)),
            pl.BlockSpec((_BLK, _NCLS), lambda i: (i + off, 0)),
        ],
        out_specs=[
            pl.BlockSpec((_BLK,), lambda i: (i,)),
            pl.BlockSpec((_BLK,), lambda i: (i,)),
        ],
        out_shape=[
            jax.ShapeDtypeStruct((_HALF,), jnp.float32),
            jax.ShapeDtypeStruct((_HALF,), jnp.int32),
        ],
    )(y_hat, y)


def _accumulate_and_merge(loss_hbm, cls_hbm, s, loss_v, cls_v, acc, big_v,
                          shared):
    def zero_chunk(i, _):
        acc[pl.ds(i * 16, 16)] = jnp.zeros((16,), jnp.float32)
        return 0
    lax.fori_loop(0, 2 * _NPAD // 16, zero_chunk, 0)

    n_per = _HALF // 16
    base = s * n_per
    pltpu.sync_copy(loss_hbm.at[pl.ds(base, n_per)], loss_v)
    pltpu.sync_copy(cls_hbm.at[pl.ds(base, n_per)], cls_v)

    ones = jnp.ones((16,), jnp.float32)

    def accum(j, _):
        lv = loss_v[pl.ds(j * 16, 16)]
        cv = cls_v[pl.ds(j * 16, 16)]
        plsc.addupdate_scatter(acc, [cv], lv)
        plsc.addupdate_scatter(acc, [cv + _NPAD], ones)
        return 0
    lax.fori_loop(0, n_per // 16, accum, 0)

    pltpu.sync_copy(acc, shared.at[s])
    plsc.subcore_barrier()
    pltpu.sync_copy(shared, big_v)


def _reduce_slice(big_v, cbase, k):
    def red(t, v):
        vs, vc = v
        vs = vs + big_v[t, pl.ds(cbase + k * 16, 16)]
        vc = vc + big_v[t, pl.ds(_NPAD + cbase + k * 16, 16)]
        return (vs, vc)
    z = jnp.zeros((16,), jnp.float32)
    return lax.fori_loop(0, 16, red, (z, z))


def _sca_body(loss_hbm, cls_hbm, out_hbm,
              loss_v, cls_v, acc, big_v, out_s, out_c, shared):
    c = lax.axis_index("c")
    s = lax.axis_index("s")

    @pl.when(c == 0)
    def _():
        _accumulate_and_merge(loss_hbm, cls_hbm, s, loss_v, cls_v, acc,
                              big_v, shared)
        cbase = s * 64
        for k in range(4):
            vs, vc = _reduce_slice(big_v, cbase, k)
            out_s[pl.ds(k * 16, 16)] = vs
            out_c[pl.ds(k * 16, 16)] = vc
        pltpu.sync_copy(out_s, out_hbm.at[pl.ds(cbase, 64)])
        pltpu.sync_copy(out_c, out_hbm.at[pl.ds(_NPAD + cbase, 64)])


def _scb_body(loss_hbm, cls_hbm, p0_hbm, out_hbm,
              loss_v, cls_v, acc, big_v, ps_v, pc_v, out_v, shared):
    c = lax.axis_index("c")
    s = lax.axis_index("s")

    @pl.when(c == 0)
    def _():
        _accumulate_and_merge(loss_hbm, cls_hbm, s, loss_v, cls_v, acc,
                              big_v, shared)
        cbase = s * 64
        pltpu.sync_copy(p0_hbm.at[pl.ds(cbase, 64)], ps_v)
        pltpu.sync_copy(p0_hbm.at[pl.ds(_NPAD + cbase, 64)], pc_v)
        for k in range(4):
            vs, vc = _reduce_slice(big_v, cbase, k)
            vs = vs + ps_v[pl.ds(k * 16, 16)]
            vc = vc + pc_v[pl.ds(k * 16, 16)]
            out_v[pl.ds(k * 16, 16)] = vs / vc
        pltpu.sync_copy(out_v, out_hbm.at[pl.ds(cbase, 64)])


_MESH = plsc.VectorSubcoreMesh(core_axis_name="c", subcore_axis_name="s")
_N_PER = _HALF // 16
_COMMON_SCRATCH = [
    pltpu.VMEM((_N_PER,), jnp.float32),
    pltpu.VMEM((_N_PER,), jnp.int32),
    pltpu.VMEM((2 * _NPAD,), jnp.float32),
    pltpu.VMEM((16, 2 * _NPAD), jnp.float32),
]


def _sc_partial(loss, cls):
    f = functools.partial(
        pl.kernel,
        mesh=_MESH,
        out_type=jax.ShapeDtypeStruct((2 * _NPAD,), jnp.float32),
        compiler_params=pltpu.CompilerParams(needs_layout_passes=False),
        scratch_types=_COMMON_SCRATCH + [
            pltpu.VMEM((64,), jnp.float32),
            pltpu.VMEM((64,), jnp.float32),
            pltpu.VMEM_SHARED((16, 2 * _NPAD), jnp.float32),
        ],
    )(_sca_body)
    return f(loss, cls)


def _sc_final(loss, cls, p0):
    f = functools.partial(
        pl.kernel,
        mesh=_MESH,
        out_type=jax.ShapeDtypeStruct((_NPAD,), jnp.float32),
        compiler_params=pltpu.CompilerParams(needs_layout_passes=False),
        scratch_types=_COMMON_SCRATCH + [
            pltpu.VMEM((64,), jnp.float32),
            pltpu.VMEM((64,), jnp.float32),
            pltpu.VMEM((64,), jnp.float32),
            pltpu.VMEM_SHARED((16, 2 * _NPAD), jnp.float32),
        ],
    )(_scb_body)
    return f(loss, cls, p0)


def kernel(y_hat, y):
    loss0, cls0 = _tc_loss(y_hat, y, 0)
    p0 = _sc_partial(loss0, cls0)
    loss1, cls1 = _tc_loss(y_hat, y, _HALF)
    out = _sc_final(loss1, cls1, p0)
    return out[:_NCLS]
